# baseline (speedup 1.0000x reference)
import functools

import jax
import jax.numpy as jnp
from jax import lax
from jax.experimental import pallas as pl
from jax.experimental.pallas import tpu as pltpu
from jax.experimental.pallas import tpu_sc as plsc

BATCH = 16384
EMBED = 64
HIDDEN = 128
NROWS = 100000
CHUNK = 128
TBLK = 16384
SPLIT = 49152
PAIRED = NROWS - SPLIT


def _transpose_body(ta_ref, tb_ref, o_ref):
    eye = jnp.eye(EMBED, dtype=jnp.float32)
    dn = (((0,), (0,)), ((), ()))
    ta = jax.lax.dot_general(ta_ref[...], eye, dn,
                             preferred_element_type=jnp.float32)
    tb = jax.lax.dot_general(tb_ref[...], eye, dn,
                             preferred_element_type=jnp.float32)
    o_ref[...] = jnp.concatenate([ta, tb], axis=1)


def _tc_transpose(t_T):
    grid = (pl.cdiv(PAIRED, TBLK),)
    return pl.pallas_call(
        _transpose_body,
        grid=grid,
        in_specs=[
            pl.BlockSpec((EMBED, TBLK), lambda i: (0, i)),
            pl.BlockSpec((EMBED, TBLK), lambda i: (0, i + SPLIT // TBLK)),
        ],
        out_specs=pl.BlockSpec((TBLK, 2 * EMBED), lambda i: (i, 0)),
        out_shape=jax.ShapeDtypeStruct((PAIRED, 2 * EMBED), jnp.float32),
    )(t_T, t_T)


def _sc_gather(table64, ids):
    info = plsc.get_sparse_core_info()
    nw = info.num_cores * info.num_subcores
    b_per_w = BATCH // nw
    n_ch = b_per_w // CHUNK

    mesh = plsc.VectorSubcoreMesh(core_axis_name="c", subcore_axis_name="s")

    @functools.partial(
        pl.kernel,
        mesh=mesh,
        out_type=jax.ShapeDtypeStruct((BATCH, 2 * EMBED), jnp.float32),
        scratch_types=[
            pltpu.VMEM((b_per_w,), jnp.int32),
            pltpu.VMEM((b_per_w, EMBED), jnp.float32),
            pltpu.SemaphoreType.DMA,
        ],
        compiler_params=pltpu.CompilerParams(use_tc_tiling_on_sc=False),
    )
    def gather_kernel(tab_hbm, ids_hbm, out_hbm, idx_v, rows_v, sem):
        wid = lax.axis_index("s") * info.num_cores + lax.axis_index("c")
        base = wid * b_per_w
        pltpu.sync_copy(ids_hbm.at[pl.ds(base, b_per_w)], idx_v)
        copies = []
        for j in range(n_ch):
            rows = pl.ds(j * CHUNK, CHUNK)
            copies.append(pltpu.async_copy(
                tab_hbm.at[idx_v.at[rows]], rows_v.at[rows], sem))
        for c in copies:
            c.wait()
        pltpu.sync_copy(rows_v,
                        out_hbm.at[pl.ds(base, b_per_w), pl.ds(0, EMBED)])

    return gather_kernel(table64, ids)


def _mlp_body(u_ref, v_ref, w1a_ref, w1b_ref, b1_ref,
              w2_ref, b2_ref, w3_ref, b3_ref, o_ref):
    u = u_ref[...][:, :EMBED]
    v = v_ref[...][:, :EMBED]
    h = jnp.dot(u, w1a_ref[...], preferred_element_type=jnp.float32)
    h = h + jnp.dot(v, w1b_ref[...], preferred_element_type=jnp.float32)
    h = jnp.maximum(h + b1_ref[...], 0.0)
    h = jnp.dot(h, w2_ref[...], preferred_element_type=jnp.float32)
    h = jnp.maximum(h + b2_ref[...], 0.0)
    o = jax.lax.dot_general(w3_ref[...], h, (((1,), (1,)), ((), ())),
                            preferred_element_type=jnp.float32)
    o = o + b3_ref[...]
    o_ref[...] = 1.0 / (1.0 + jnp.exp(-o))


def _tc_mlp(u, v, w1a_t, w1b_t, b1r, w2_t, b2r, w3, b3r):
    bb = 2048
    grid = (BATCH // bb,)
    full = lambda shape: pl.BlockSpec(shape, lambda i: (0, 0))
    return pl.pallas_call(
        _mlp_body,
        grid=grid,
        in_specs=[
            pl.BlockSpec((bb, 2 * EMBED), lambda i: (i, 0)),
            pl.BlockSpec((bb, 2 * EMBED), lambda i: (i, 0)),
            full((EMBED, HIDDEN)),
            full((EMBED, HIDDEN)),
            full((1, HIDDEN)),
            full((HIDDEN, HIDDEN // 2)),
            full((1, HIDDEN // 2)),
            full((1, HIDDEN // 2)),
            full((1, 1)),
        ],
        out_specs=pl.BlockSpec((1, bb), lambda i: (0, i)),
        out_shape=jax.ShapeDtypeStruct((1, BATCH), jnp.float32),
    )(u, v, w1a_t, w1b_t, b1r, w2_t, b2r, w3, b3r)


def kernel(user_ids, item_ids, user_table, item_table, W1, b1, W2, b2, W3, b3):
    utp = _tc_transpose(user_table.T).reshape(2 * PAIRED, EMBED)
    itp = _tc_transpose(item_table.T).reshape(2 * PAIRED, EMBED)
    uidx = jnp.where(user_ids < SPLIT, 2 * user_ids,
                     2 * (user_ids - SPLIT) + 1)
    iidx = jnp.where(item_ids < SPLIT, 2 * item_ids,
                     2 * (item_ids - SPLIT) + 1)
    u = _sc_gather(utp, uidx)
    v = _sc_gather(itp, iidx)

    out = _tc_mlp(u, v, W1[:, :EMBED].T, W1[:, EMBED:].T,
                  b1.reshape(1, -1), W2.T, b2.reshape(1, -1), W3,
                  b3.reshape(1, 1))
    return out.reshape(BATCH, 1)


# speedup vs baseline: 1.1059x; 1.1059x over previous
import functools

import jax
import jax.numpy as jnp
from jax import lax
from jax.experimental import pallas as pl
from jax.experimental.pallas import tpu as pltpu
from jax.experimental.pallas import tpu_sc as plsc

BATCH = 16384
EMBED = 64
HIDDEN = 128
NROWS = 100000
CHUNK = 128
TBLK = 8192
SPLIT = 49152
PAIRED = NROWS - SPLIT


def _transpose_body(ta_ref, tb_ref, o_ref):
    eye = jnp.eye(EMBED, dtype=jnp.float32)
    dn = (((0,), (0,)), ((), ()))
    ta = jax.lax.dot_general(ta_ref[...], eye, dn,
                             preferred_element_type=jnp.float32)
    tb = jax.lax.dot_general(tb_ref[...], eye, dn,
                             preferred_element_type=jnp.float32)
    o_ref[...] = jnp.concatenate([ta, tb], axis=1)


def _tc_transpose(t_T):
    grid = (pl.cdiv(PAIRED, TBLK),)
    return pl.pallas_call(
        _transpose_body,
        grid=grid,
        in_specs=[
            pl.BlockSpec((EMBED, TBLK), lambda i: (0, i)),
            pl.BlockSpec((EMBED, TBLK), lambda i: (0, i + SPLIT // TBLK)),
        ],
        out_specs=pl.BlockSpec((TBLK, 2 * EMBED), lambda i: (i, 0)),
        out_shape=jax.ShapeDtypeStruct((PAIRED, 2 * EMBED), jnp.float32),
    )(t_T, t_T)


def _sc_gather(table64, ids):
    info = plsc.get_sparse_core_info()
    nw = info.num_cores * info.num_subcores
    b_per_w = BATCH // nw
    n_ch = b_per_w // CHUNK

    mesh = plsc.VectorSubcoreMesh(core_axis_name="c", subcore_axis_name="s")

    @functools.partial(
        pl.kernel,
        mesh=mesh,
        out_type=jax.ShapeDtypeStruct((BATCH, 2 * EMBED), jnp.float32),
        scratch_types=[
            pltpu.VMEM((b_per_w,), jnp.int32),
            pltpu.VMEM((b_per_w, EMBED), jnp.float32),
            pltpu.SemaphoreType.DMA,
        ],
        compiler_params=pltpu.CompilerParams(use_tc_tiling_on_sc=False),
    )
    def gather_kernel(tab_hbm, ids_hbm, out_hbm, idx_v, rows_v, sem):
        wid = lax.axis_index("s") * info.num_cores + lax.axis_index("c")
        base = wid * b_per_w
        pltpu.sync_copy(ids_hbm.at[pl.ds(base, b_per_w)], idx_v)
        copies = []
        for j in range(n_ch):
            rows = pl.ds(j * CHUNK, CHUNK)
            copies.append(pltpu.async_copy(
                tab_hbm.at[idx_v.at[rows]], rows_v.at[rows], sem))
        for c in copies:
            c.wait()
        pltpu.sync_copy(rows_v,
                        out_hbm.at[pl.ds(base, b_per_w), pl.ds(0, EMBED)])

    return gather_kernel(table64, ids)


def _mlp_body(u_ref, v_ref, w1a_ref, w1b_ref, b1_ref,
              w2_ref, b2_ref, w3_ref, b3_ref, o_ref):
    u = u_ref[...][:, :EMBED]
    v = v_ref[...][:, :EMBED]
    h = jnp.dot(u, w1a_ref[...], preferred_element_type=jnp.float32)
    h = h + jnp.dot(v, w1b_ref[...], preferred_element_type=jnp.float32)
    h = jnp.maximum(h + b1_ref[...], 0.0)
    h = jnp.dot(h, w2_ref[...], preferred_element_type=jnp.float32)
    h = jnp.maximum(h + b2_ref[...], 0.0)
    o = jax.lax.dot_general(w3_ref[...], h, (((1,), (1,)), ((), ())),
                            preferred_element_type=jnp.float32)
    o = o + b3_ref[...]
    o_ref[...] = 1.0 / (1.0 + jnp.exp(-o))


def _tc_mlp(u, v, w1a_t, w1b_t, b1r, w2_t, b2r, w3, b3r):
    bb = 4096
    grid = (BATCH // bb,)
    full = lambda shape: pl.BlockSpec(shape, lambda i: (0, 0))
    return pl.pallas_call(
        _mlp_body,
        grid=grid,
        in_specs=[
            pl.BlockSpec((bb, 2 * EMBED), lambda i: (i, 0)),
            pl.BlockSpec((bb, 2 * EMBED), lambda i: (i, 0)),
            full((EMBED, HIDDEN)),
            full((EMBED, HIDDEN)),
            full((1, HIDDEN)),
            full((HIDDEN, HIDDEN // 2)),
            full((1, HIDDEN // 2)),
            full((1, HIDDEN // 2)),
            full((1, 1)),
        ],
        out_specs=pl.BlockSpec((1, bb), lambda i: (0, i)),
        out_shape=jax.ShapeDtypeStruct((1, BATCH), jnp.float32),
    )(u, v, w1a_t, w1b_t, b1r, w2_t, b2r, w3, b3r)


def kernel(user_ids, item_ids, user_table, item_table, W1, b1, W2, b2, W3, b3):
    utp = _tc_transpose(user_table.T).reshape(2 * PAIRED, EMBED)
    itp = _tc_transpose(item_table.T).reshape(2 * PAIRED, EMBED)
    uidx = jnp.where(user_ids < SPLIT, 2 * user_ids,
                     2 * (user_ids - SPLIT) + 1)
    iidx = jnp.where(item_ids < SPLIT, 2 * item_ids,
                     2 * (item_ids - SPLIT) + 1)
    u = _sc_gather(utp, uidx)
    v = _sc_gather(itp, iidx)

    out = _tc_mlp(u, v, W1[:, :EMBED].T, W1[:, EMBED:].T,
                  b1.reshape(1, -1), W2.T, b2.reshape(1, -1), W3,
                  b3.reshape(1, 1))
    return out.reshape(BATCH, 1)
